# baseline (speedup 1.0000x reference)
import functools

import jax
import jax.numpy as jnp
from jax.experimental import pallas as pl
from jax.experimental.pallas import tpu as pltpu

JBLK = 2048


def _ln_rows(v, g, b, eps=1e-5):
    m = jnp.mean(v, axis=-1, keepdims=True)
    var = jnp.mean((v - m) ** 2, axis=-1, keepdims=True)
    return (v - m) * jax.lax.rsqrt(var + eps) * g + b


def _main_kernel(
    xT_ref, maskT_ref, f_ref, ae_ref_in, g1aT_ref,
    w1x_ref, hb1_ref, h1g_ref, h1b_ref, w1fT_ref,
    w2T_ref, hb2_ref, h2g_ref, h2b_ref,
    g1hT_ref, gb1_ref, g2T_ref, gb2_ref,
    acc_out_ref, s_out_ref,
    fw_ref, aeg_ref, s_ref, acc_ref,
    *, nj, wheads,
):
    j = pl.program_id(0)
    b = pl.program_id(1)
    f32 = jnp.float32

    @pl.when(b == 0)
    def _():
        fw_ref[...] = jnp.dot(f_ref[...], w1fT_ref[...], preferred_element_type=f32)
        aeg_ref[...] = jnp.dot(ae_ref_in[...], g1aT_ref[...], preferred_element_type=f32)

    @pl.when(j == 0)
    def _():
        s_ref[b] = jnp.zeros((1, s_ref.shape[2]), f32)
        acc_ref[b] = jnp.zeros(acc_ref.shape[1:], f32)

    xcol = xT_ref[0]
    obs = maskT_ref[0] > 0

    h1 = fw_ref[...] + xcol * w1x_ref[...] + hb1_ref[...]
    h = jnp.maximum(_ln_rows(h1, h1g_ref[...], h1b_ref[...]), 0.0)
    h2 = jnp.dot(h, w2T_ref[...], preferred_element_type=f32) + hb2_ref[...]
    hout = jnp.maximum(_ln_rows(h2, h2g_ref[...], h2b_ref[...]), 0.0)

    g = jnp.dot(hout, g1hT_ref[...], preferred_element_type=f32)
    g = jnp.maximum(g + aeg_ref[...] + gb1_ref[...], 0.0)
    raw = jnp.dot(g, g2T_ref[...], preferred_element_type=f32) + gb2_ref[...]
    logits = jnp.clip(raw, -10.0, 10.0)
    p = jnp.where(obs, jnp.exp(logits - 10.0), 0.0)

    s_ref[b] += jnp.sum(p, axis=0, keepdims=True)
    acc_ref[b] += jax.lax.dot_general(
        p, hout, (((0,), (0,)), ((), ())), preferred_element_type=f32)

    @pl.when(j == nj - 1)
    def _():
        acc_out_ref[...] = acc_ref[b].reshape(1, wheads, acc_ref.shape[2])
        s_out_ref[...] = s_ref[b].reshape(1, 1, wheads)


def _tail_kernel(
    acc_ref, s_ref,
    cWT_ref, cb_ref, cg_ref, cbeta_ref,
    eW1T_ref, eb1_ref, e1g_ref, e1b_ref,
    eW2T_ref, eb2_ref, e2g_ref, e2b_ref,
    mu_ref, lv_ref,
    *, wheads, lhalf,
):
    f32 = jnp.float32
    s = s_ref[:, 0, :]
    inv = jnp.where(s > 0, 1.0 / s, 0.0)
    d = acc_ref.shape[2]
    combined = cb_ref[...]
    for w in range(wheads):
        hs_w = acc_ref[:, w, :] * inv[:, w:w + 1]
        combined = combined + jnp.dot(
            hs_w, cWT_ref[w * d:(w + 1) * d, :], preferred_element_type=f32)
    combined = jnp.maximum(_ln_rows(combined, cg_ref[...], cbeta_ref[...]), 0.0)
    has = s[:, 0:1] > 0
    c = jnp.where(has, combined, 0.0)
    z = jnp.dot(c, eW1T_ref[...], preferred_element_type=f32) + eb1_ref[...]
    z = jnp.maximum(_ln_rows(z, e1g_ref[...], e1b_ref[...]), 0.0)
    ml = jnp.dot(z, eW2T_ref[...], preferred_element_type=f32) + eb2_ref[...]
    ml = jnp.maximum(_ln_rows(ml, e2g_ref[...], e2b_ref[...]), 0.0)
    mu_ref[...] = ml[:, :lhalf]
    lv_ref[...] = ml[:, lhalf:]


def kernel(x, mask, feature_embedding, atse_index_per_j, atse_embedding,
           h_W1, h_b1, h_ln1_g, h_ln1_b, h_W2, h_b2, h_ln2_g, h_ln2_b,
           g_W1, g_b1, g_W2, g_b2, c_W, c_b, c_ln_g, c_ln_b,
           e_W1, e_b1, e_ln1_g, e_ln1_b, e_W2, e_b2, e_ln2_g, e_ln2_b):
    B, J = x.shape
    D = feature_embedding.shape[1]
    H = h_W1.shape[0]
    gh = g_W1.shape[0]
    W = g_W2.shape[0]
    L2 = e_W2.shape[0]
    L = L2 // 2
    nj = J // JBLK
    f32 = jnp.float32

    x3 = x.reshape(B, J, 1)
    mask3 = mask.reshape(B, J, 1)
    ae_all = atse_embedding[atse_index_per_j]
    row = lambda v: v.reshape(1, -1)
    w1x = row(h_W1[:, 0])
    w1fT = h_W1[:, 1:].T
    w2T = h_W2.T
    g1hT = g_W1[:, :D].T
    g1aT = g_W1[:, D:].T
    g2T = g_W2.T
    cWT = c_W.T
    eW1T = e_W1.T
    eW2T = e_W2.T

    jblk_spec = lambda n: pl.BlockSpec((JBLK, n), lambda j, b: (j, 0))
    full = lambda a: pl.BlockSpec(a.shape, lambda j, b: (0,) * a.ndim)

    ins = [
        x3, mask3, feature_embedding, ae_all, g1aT,
        w1x, row(h_b1), row(h_ln1_g), row(h_ln1_b), w1fT,
        w2T, row(h_b2), row(h_ln2_g), row(h_ln2_b),
        g1hT, row(g_b1), g2T, row(g_b2),
    ]
    col_spec = pl.BlockSpec((1, JBLK, 1), lambda j, b: (b, j, 0))
    in_specs = [
        col_spec, col_spec, jblk_spec(D), jblk_spec(ae_all.shape[1]),
    ] + [full(a) for a in ins[4:]]

    acc, s = pl.pallas_call(
        functools.partial(_main_kernel, nj=nj, wheads=W),
        grid=(nj, B),
        in_specs=in_specs,
        out_specs=[
            pl.BlockSpec((1, W, D), lambda j, b: (b, 0, 0)),
            pl.BlockSpec((1, 1, W), lambda j, b: (b, 0, 0)),
        ],
        out_shape=[
            jax.ShapeDtypeStruct((B, W, D), f32),
            jax.ShapeDtypeStruct((B, 1, W), f32),
        ],
        scratch_shapes=[
            pltpu.VMEM((JBLK, H), f32),
            pltpu.VMEM((JBLK, gh), f32),
            pltpu.VMEM((B, 1, W), f32),
            pltpu.VMEM((B, W, D), f32),
        ],
    )(*ins)

    tail_ins = [
        acc, s,
        cWT, row(c_b), row(c_ln_g), row(c_ln_b),
        eW1T, row(e_b1), row(e_ln1_g), row(e_ln1_b),
        eW2T, row(e_b2), row(e_ln2_g), row(e_ln2_b),
    ]
    mu, lv = pl.pallas_call(
        functools.partial(_tail_kernel, wheads=W, lhalf=L),
        in_specs=[pl.BlockSpec(a.shape, functools.partial(lambda n: (0,) * n, a.ndim))
                  for a in tail_ins],
        out_specs=[pl.BlockSpec((B, L), lambda: (0, 0))] * 2,
        out_shape=[jax.ShapeDtypeStruct((B, L), f32)] * 2,
    )(*tail_ins)
    return (mu, lv)


# speedup vs baseline: 1.1245x; 1.1245x over previous
import functools

import jax
import jax.numpy as jnp
from jax.experimental import pallas as pl
from jax.experimental.pallas import tpu as pltpu

JBLK = 4096


def _ln_rows(v, g, b, eps=1e-5):
    m = jnp.mean(v, axis=-1, keepdims=True)
    var = jnp.mean((v - m) ** 2, axis=-1, keepdims=True)
    return (v - m) * jax.lax.rsqrt(var + eps) * g + b


def _main_kernel(
    xT_ref, maskT_ref, f_ref, ae_ref_in, g1aT_ref,
    wt_ref, c2_ref, hb1_ref, h1g_ref, h1b_ref, w1fT_ref,
    w2T_ref, hb2_ref, h2g_ref, h2b_ref,
    g1hT_ref, gb1_ref, g2T_ref, gb2_ref,
    acc_out_ref, s_out_ref,
    fw_ref, aeg_ref, c0_ref, c1_ref, s_ref, acc_ref,
    *, nj, wheads,
):
    j = pl.program_id(0)
    b = pl.program_id(1)
    f32 = jnp.float32

    @pl.when(b == 0)
    def _():
        fwraw = jnp.dot(f_ref[...], w1fT_ref[...],
                        preferred_element_type=f32) + hb1_ref[...]
        u = fwraw - jnp.mean(fwraw, axis=1, keepdims=True)
        fw_ref[...] = u
        c0_ref[...] = jnp.mean(u * u, axis=1, keepdims=True)
        c1_ref[...] = 2.0 * jnp.mean(u * wt_ref[...], axis=1, keepdims=True)
        aeg_ref[...] = jnp.dot(ae_ref_in[...], g1aT_ref[...],
                               preferred_element_type=f32) + gb1_ref[...]

    @pl.when(j == 0)
    def _():
        s_ref[b] = jnp.zeros((1, s_ref.shape[2]), f32)
        acc_ref[b] = jnp.zeros(acc_ref.shape[1:], f32)

    xcol = xT_ref[0]
    obs = maskT_ref[0] > 0

    h1c = fw_ref[...] + xcol * wt_ref[...]
    var = (c0_ref[...] + xcol * c1_ref[...]
           + (xcol * xcol) * c2_ref[...])
    a = jax.lax.rsqrt(var + 1e-5)
    h = jnp.maximum((h1c * a) * h1g_ref[...] + h1b_ref[...], 0.0)
    h2 = jnp.dot(h, w2T_ref[...], preferred_element_type=f32) + hb2_ref[...]
    hout = jnp.maximum(_ln_rows(h2, h2g_ref[...], h2b_ref[...]), 0.0)

    g = jnp.dot(hout, g1hT_ref[...], preferred_element_type=f32)
    g = jnp.maximum(g + aeg_ref[...], 0.0)
    raw = jnp.dot(g, g2T_ref[...], preferred_element_type=f32) + gb2_ref[...]
    logits = jnp.clip(raw, -10.0, 10.0)
    p = jnp.where(obs, jnp.exp(logits - 10.0), 0.0)

    s_ref[b] += jnp.sum(p, axis=0, keepdims=True)
    acc_ref[b] += jax.lax.dot_general(
        p, hout, (((0,), (0,)), ((), ())), preferred_element_type=f32)

    @pl.when(j == nj - 1)
    def _():
        acc_out_ref[...] = acc_ref[b].reshape(1, wheads, acc_ref.shape[2])
        s_out_ref[...] = s_ref[b].reshape(1, 1, wheads)


def _tail_kernel(
    acc_ref, s_ref,
    cWT_ref, cb_ref, cg_ref, cbeta_ref,
    eW1T_ref, eb1_ref, e1g_ref, e1b_ref,
    eW2T_ref, eb2_ref, e2g_ref, e2b_ref,
    mu_ref, lv_ref,
    *, wheads, lhalf,
):
    f32 = jnp.float32
    s = s_ref[:, 0, :]
    inv = jnp.where(s > 0, 1.0 / s, 0.0)
    d = acc_ref.shape[2]
    combined = cb_ref[...]
    for w in range(wheads):
        hs_w = acc_ref[:, w, :] * inv[:, w:w + 1]
        combined = combined + jnp.dot(
            hs_w, cWT_ref[w * d:(w + 1) * d, :], preferred_element_type=f32)
    combined = jnp.maximum(_ln_rows(combined, cg_ref[...], cbeta_ref[...]), 0.0)
    has = s[:, 0:1] > 0
    c = jnp.where(has, combined, 0.0)
    z = jnp.dot(c, eW1T_ref[...], preferred_element_type=f32) + eb1_ref[...]
    z = jnp.maximum(_ln_rows(z, e1g_ref[...], e1b_ref[...]), 0.0)
    ml = jnp.dot(z, eW2T_ref[...], preferred_element_type=f32) + eb2_ref[...]
    ml = jnp.maximum(_ln_rows(ml, e2g_ref[...], e2b_ref[...]), 0.0)
    mu_ref[...] = ml[:, :lhalf]
    lv_ref[...] = ml[:, lhalf:]


def kernel(x, mask, feature_embedding, atse_index_per_j, atse_embedding,
           h_W1, h_b1, h_ln1_g, h_ln1_b, h_W2, h_b2, h_ln2_g, h_ln2_b,
           g_W1, g_b1, g_W2, g_b2, c_W, c_b, c_ln_g, c_ln_b,
           e_W1, e_b1, e_ln1_g, e_ln1_b, e_W2, e_b2, e_ln2_g, e_ln2_b):
    B, J = x.shape
    D = feature_embedding.shape[1]
    H = h_W1.shape[0]
    gh = g_W1.shape[0]
    W = g_W2.shape[0]
    L2 = e_W2.shape[0]
    L = L2 // 2
    nj = J // JBLK
    f32 = jnp.float32

    x3 = x.reshape(B, J, 1)
    mask3 = mask.reshape(B, J, 1)
    ae_all = atse_embedding[atse_index_per_j]
    row = lambda v: v.reshape(1, -1)
    w1x = row(h_W1[:, 0])
    wt = w1x - jnp.mean(w1x)
    c2 = jnp.mean(wt * wt).reshape(1, 1)
    w1fT = h_W1[:, 1:].T
    w2T = h_W2.T
    g1hT = g_W1[:, :D].T
    g1aT = g_W1[:, D:].T
    g2T = g_W2.T
    cWT = c_W.T
    eW1T = e_W1.T
    eW2T = e_W2.T

    jblk_spec = lambda n: pl.BlockSpec((JBLK, n), lambda j, b: (j, 0))
    full = lambda a: pl.BlockSpec(a.shape, lambda j, b: (0,) * a.ndim)

    ins = [
        x3, mask3, feature_embedding, ae_all, g1aT,
        wt, c2, row(h_b1), row(h_ln1_g), row(h_ln1_b), w1fT,
        w2T, row(h_b2), row(h_ln2_g), row(h_ln2_b),
        g1hT, row(g_b1), g2T, row(g_b2),
    ]
    col_spec = pl.BlockSpec((1, JBLK, 1), lambda j, b: (b, j, 0))
    in_specs = [
        col_spec, col_spec, jblk_spec(D), jblk_spec(ae_all.shape[1]),
    ] + [full(a) for a in ins[4:]]

    acc, s = pl.pallas_call(
        functools.partial(_main_kernel, nj=nj, wheads=W),
        grid=(nj, B),
        in_specs=in_specs,
        out_specs=[
            pl.BlockSpec((1, W, D), lambda j, b: (b, 0, 0)),
            pl.BlockSpec((1, 1, W), lambda j, b: (b, 0, 0)),
        ],
        out_shape=[
            jax.ShapeDtypeStruct((B, W, D), f32),
            jax.ShapeDtypeStruct((B, 1, W), f32),
        ],
        scratch_shapes=[
            pltpu.VMEM((JBLK, H), f32),
            pltpu.VMEM((JBLK, gh), f32),
            pltpu.VMEM((JBLK, 1), f32),
            pltpu.VMEM((JBLK, 1), f32),
            pltpu.VMEM((B, 1, W), f32),
            pltpu.VMEM((B, W, D), f32),
        ],
    )(*ins)

    tail_ins = [
        acc, s,
        cWT, row(c_b), row(c_ln_g), row(c_ln_b),
        eW1T, row(e_b1), row(e_ln1_g), row(e_ln1_b),
        eW2T, row(e_b2), row(e_ln2_g), row(e_ln2_b),
    ]
    mu, lv = pl.pallas_call(
        functools.partial(_tail_kernel, wheads=W, lhalf=L),
        in_specs=[pl.BlockSpec(a.shape, functools.partial(lambda n: (0,) * n, a.ndim))
                  for a in tail_ins],
        out_specs=[pl.BlockSpec((B, L), lambda: (0, 0))] * 2,
        out_shape=[jax.ShapeDtypeStruct((B, L), f32)] * 2,
    )(*tail_ins)
    return (mu, lv)
